# baseline (speedup 1.0000x reference)
import functools

import jax
import jax.numpy as jnp
from jax import lax
from jax.experimental import pallas as pl
from jax.experimental.pallas import tpu as pltpu
from jax.experimental.pallas import tpu_sc as plsc

_R, _C = 500, 20000
_NC, _NS = 2, 16
_NW = _NC * _NS
_W = 640
_NH = 2
_HK = 20
_H = _HK * 16
_BR = 50
_NRB = _R // _BR
_RPAD = 512

_LOW = 0.3
_HIGH = 0.7

_mesh = plsc.VectorSubcoreMesh(core_axis_name="c", subcore_axis_name="s")


def _hmax16(v):
    dnums = lax.GatherDimensionNumbers(
        offset_dims=(), collapsed_slice_dims=(0,), start_index_map=(0,))
    for d in (8, 4, 2, 1):
        idx = lax.iota(jnp.int32, 16) ^ d
        perm = lax.gather(v, idx[:, None], dnums, (1,),
                          mode=lax.GatherScatterMode.PROMISE_IN_BOUNDS)
        v = jnp.maximum(v, perm)
    return v


def _worker():
    wid = lax.axis_index("s") * _NC + lax.axis_index("c")
    cstart = (wid * 625) // 16 * 16
    return cstart, wid


def _pass1_body(x, cmax_out, cam_out, rmaxp_out,
                buf, cmaxb, camb, rmaxb, accrow, sems):
    cstart, wid = _worker()

    copies = [None] * _NRB

    def start(rb):
        cp = pltpu.make_async_copy(
            x.at[pl.ds(rb * _BR, _BR), pl.ds(cstart, _W)],
            buf.at[rb % 2],
            sems.at[rb % 2],
        )
        cp.start()
        copies[rb] = cp

    start(0)
    if _NRB > 1:
        start(1)

    cmx = [jnp.full((16,), -1.0, jnp.float32)] * (_NH * _HK)
    cam = [jnp.full((16,), 0, jnp.int32)] * (_NH * _HK)

    for rb in range(_NRB):
        slot = rb % 2
        copies[rb].wait()
        for h in range(_NH):
            def rowbody(r, cy, rb=rb, h=h, slot=slot):
                cms, ams = cy
                rg = rb * _BR + r
                rs = jnp.full((16,), rg, jnp.int32)
                vs = [buf[slot, r, pl.ds(h * _H + 16 * k, 16)]
                      for k in range(_HK)]
                ncm, nam = [], []
                for k in range(_HK):
                    m = vs[k] > cms[k]
                    ncm.append(jnp.where(m, vs[k], cms[k]))
                    nam.append(jnp.where(m, rs, ams[k]))
                acc = functools.reduce(jnp.maximum, vs)
                if h == 0:
                    accrow[r, :] = acc
                else:
                    hm = _hmax16(jnp.maximum(accrow[r, :], acc))
                    plsc.store_scatter(
                        rmaxb, [rs], hm,
                        mask=lax.iota(jnp.int32, 16) == 0)
                return (tuple(ncm), tuple(nam))

            carry = (tuple(cmx[h * _HK:(h + 1) * _HK]),
                     tuple(cam[h * _HK:(h + 1) * _HK]))
            ncm, nam = lax.fori_loop(0, _BR, rowbody, carry)
            cmx[h * _HK:(h + 1) * _HK] = list(ncm)
            cam[h * _HK:(h + 1) * _HK] = list(nam)
        if rb + 2 < _NRB:
            start(rb + 2)

    for h in range(_NH):
        for k in range(_HK):
            cmaxb[pl.ds(h * _H + 16 * k, 16)] = cmx[h * _HK + k]
            camb[pl.ds(h * _H + 16 * k, 16)] = cam[h * _HK + k]
    pltpu.sync_copy(cmaxb, cmax_out.at[pl.ds(cstart, _W)])
    pltpu.sync_copy(camb, cam_out.at[pl.ds(cstart, _W)])
    pltpu.sync_copy(rmaxb, rmaxp_out.at[wid])


_pass1 = functools.partial(
    pl.kernel,
    out_type=[
        jax.ShapeDtypeStruct((_C,), jnp.float32),
        jax.ShapeDtypeStruct((_C,), jnp.int32),
        jax.ShapeDtypeStruct((_NW, _RPAD), jnp.float32),
    ],
    mesh=_mesh,
    compiler_params=pltpu.CompilerParams(use_tc_tiling_on_sc=False, needs_layout_passes=False),
    scratch_types=[
        pltpu.VMEM((2, _BR, _W), jnp.float32),
        pltpu.VMEM((_W,), jnp.float32),
        pltpu.VMEM((_W,), jnp.int32),
        pltpu.VMEM((_RPAD,), jnp.float32),
        pltpu.VMEM((_BR, 16), jnp.float32),
        pltpu.SemaphoreType.DMA((2,)),
    ],
)(_pass1_body)


def _pass2_body(x, cmax_in, cam_in, rmaxp_in, out,
                buf, rmp, rmaxb, cmaxb, camb, outb, sems):
    cstart, wid = _worker()

    copies = [None] * _NRB

    def start(rb):
        cp = pltpu.make_async_copy(
            x.at[pl.ds(rb * _BR, _BR), pl.ds(cstart, _W)],
            buf.at[rb % 2],
            sems.at[rb % 2],
        )
        cp.start()
        copies[rb] = cp

    start(0)
    if _NRB > 1:
        start(1)

    pltpu.sync_copy(rmaxp_in, rmp)

    def redbody(k, _):
        def inner(j, acc):
            return jnp.maximum(acc, rmp[j, pl.ds(k * 16, 16)])
        acc = lax.fori_loop(1, _NW, inner, rmp[0, pl.ds(k * 16, 16)])
        rmaxb[pl.ds(k * 16, 16)] = acc
        return 0

    lax.fori_loop(0, _RPAD // 16, redbody, 0)

    upd = [jnp.full((16,), 0, jnp.int32)] * (_NH * _HK)
    one = jnp.full((16,), 1, jnp.int32)

    for rb in range(_NRB):
        slot = rb % 2
        copies[rb].wait()
        for h in range(_NH):
            def rowbody(r, cy, rb=rb, h=h, slot=slot):
                rg = rb * _BR + r
                rmv = plsc.load_gather(
                    rmaxb, [jnp.full((16,), rg, jnp.int32)])
                nup = []
                for k in range(_HK):
                    v = buf[slot, r, pl.ds(h * _H + 16 * k, 16)]
                    nup.append(jnp.where(v == rmv, one, cy[k]))
                return tuple(nup)

            carry = tuple(upd[h * _HK:(h + 1) * _HK])
            res = lax.fori_loop(0, _BR, rowbody, carry)
            upd[h * _HK:(h + 1) * _HK] = list(res)
        if rb + 2 < _NRB:
            start(rb + 2)

    pltpu.sync_copy(cmax_in.at[pl.ds(cstart, _W)], cmaxb)
    pltpu.sync_copy(cam_in.at[pl.ds(cstart, _W)], camb)
    for h in range(_NH):
        for k in range(_HK):
            sl = pl.ds(h * _H + 16 * k, 16)
            cm = cmaxb[sl]
            am = camb[sl]
            m = jnp.where(cm < _LOW, jnp.int32(-1),
                          jnp.where(cm < _HIGH, jnp.int32(-2), am))
            outb[sl] = jnp.where(upd[h * _HK + k] > 0, am, m)
    pltpu.sync_copy(outb, out.at[pl.ds(cstart, _W)])


_pass2 = functools.partial(
    pl.kernel,
    out_type=jax.ShapeDtypeStruct((_C,), jnp.int32),
    mesh=_mesh,
    compiler_params=pltpu.CompilerParams(use_tc_tiling_on_sc=False, needs_layout_passes=False),
    scratch_types=[
        pltpu.VMEM((2, _BR, _W), jnp.float32),
        pltpu.VMEM((_NW, _RPAD), jnp.float32),
        pltpu.VMEM((_RPAD,), jnp.float32),
        pltpu.VMEM((_W,), jnp.float32),
        pltpu.VMEM((_W,), jnp.int32),
        pltpu.VMEM((_W,), jnp.int32),
        pltpu.SemaphoreType.DMA((2,)),
    ],
)(_pass2_body)


def kernel(match_quality_matrix):
    cmax, cam, rmaxp = _pass1(match_quality_matrix)
    return _pass2(match_quality_matrix, cmax, cam, rmaxp)


# speedup vs baseline: 6.1610x; 6.1610x over previous
import jax
import jax.numpy as jnp
from jax.experimental import pallas as pl
from jax.experimental.pallas import tpu as pltpu

_R, _C = 500, 20000
_CW = 2048
_NFULL = 9
_TAILW = _C - _NFULL * _CW
_NCH = _NFULL + 1

_LOW = 0.3
_HIGH = 0.7


def _body(x_hbm, out_ref, buf, tail, cmax_ref, cam_ref, rmax_ref, sems):
    def chunk_src(k):
        ofs = k * _CW
        if k < _NFULL:
            return ofs, _CW, buf.at[:, pl.ds(ofs, _CW)]
        return ofs, _TAILW, tail.at[:, :]

    copies = []
    for k in range(_NCH):
        ofs, w, dst = chunk_src(k)
        cp = pltpu.make_async_copy(x_hbm.at[:, pl.ds(ofs, w)], dst, sems.at[k])
        cp.start()
        copies.append(cp)

    def chunk_blk(k):
        ofs, w, _ = chunk_src(k)
        if k < _NFULL:
            return ofs, w, buf[:, pl.ds(ofs, w)]
        return ofs, w, tail[:, :]

    for k in range(_NCH):
        copies[k].wait()
        ofs, w, blk = chunk_blk(k)
        part_rm = jnp.max(blk, axis=1, keepdims=True)
        if k == 0:
            rmax_ref[...] = part_rm
        else:
            rmax_ref[...] = jnp.maximum(rmax_ref[...], part_rm)
        cmax = jnp.max(blk, axis=0)
        rows = jax.lax.broadcasted_iota(jnp.int32, (_R, w), 0)
        cam = jnp.min(jnp.where(blk == cmax[None, :], rows, _R), axis=0)
        cmax_ref[0, pl.ds(ofs, w)] = cmax
        cam_ref[0, pl.ds(ofs, w)] = cam

    rm = rmax_ref[...]
    for k in range(_NCH):
        ofs, w, blk = chunk_blk(k)
        cmax = cmax_ref[0, pl.ds(ofs, w)]
        cam = cam_ref[0, pl.ds(ofs, w)]
        low = cmax < _HIGH
        m = jnp.where(cmax < _LOW, jnp.int32(-1),
                      jnp.where(low, jnp.int32(-2), cam))
        out_ref[pl.ds(ofs, w)] = m

        @pl.when(jnp.any(low))
        def _(ofs=ofs, w=w, blk=blk, cam=cam, m=m):
            upd = jnp.any(blk == rm, axis=0)
            out_ref[pl.ds(ofs, w)] = jnp.where(upd, cam, m)


def kernel(match_quality_matrix):
    return pl.pallas_call(
        _body,
        out_shape=jax.ShapeDtypeStruct((_C,), jnp.int32),
        in_specs=[pl.BlockSpec(memory_space=pl.ANY)],
        out_specs=pl.BlockSpec(memory_space=pltpu.VMEM),
        scratch_shapes=[
            pltpu.VMEM((_R, _NFULL * _CW), jnp.float32),
            pltpu.VMEM((_R, _TAILW), jnp.float32),
            pltpu.VMEM((1, _C), jnp.float32),
            pltpu.VMEM((1, _C), jnp.int32),
            pltpu.VMEM((_R, 1), jnp.float32),
            pltpu.SemaphoreType.DMA((_NCH,)),
        ],
        compiler_params=pltpu.CompilerParams(
            vmem_limit_bytes=100 * 1024 * 1024,
        ),
    )(match_quality_matrix)
